# baseline (speedup 1.0000x reference)
import jax
import jax.numpy as jnp
from jax.experimental import pallas as pl
from jax.experimental.pallas import tpu as pltpu

_BLOCK_ROWS = 3336


def _lin_swish_kernel(x_ref, w_ref, b_ref, o_ref):
    y = jax.lax.dot_general(
        x_ref[...].astype(jnp.bfloat16),
        w_ref[...].astype(jnp.bfloat16),
        dimension_numbers=(((1,), (1,)), ((), ())),
        preferred_element_type=jnp.float32,
    )
    y = y + b_ref[...]
    o_ref[...] = y * jax.nn.sigmoid(y)


def kernel(x, feature1, feature2, edge_index, params):
    del feature1, feature2, edge_index
    n, h = x.shape
    w = params["lin_w"]
    b = params["lin_b"].reshape(1, h)
    block = min(_BLOCK_ROWS, n)
    return pl.pallas_call(
        _lin_swish_kernel,
        grid=(pl.cdiv(n, block),),
        in_specs=[
            pl.BlockSpec((block, h), lambda i: (i, 0)),
            pl.BlockSpec((h, h), lambda i: (0, 0)),
            pl.BlockSpec((1, h), lambda i: (0, 0)),
        ],
        out_specs=pl.BlockSpec((block, h), lambda i: (i, 0)),
        out_shape=jax.ShapeDtypeStruct((n, h), jnp.float32),
        compiler_params=pltpu.CompilerParams(
            dimension_semantics=("parallel",),
        ),
    )(x, w, b)


# speedup vs baseline: 1.2910x; 1.2910x over previous
import jax
import jax.numpy as jnp
from jax.experimental import pallas as pl
from jax.experimental.pallas import tpu as pltpu

_BLOCK_ROWS = 5000


def _lin_swish_kernel(x_ref, w_ref, b_ref, o_ref):
    y = jax.lax.dot_general(
        x_ref[...].astype(jnp.bfloat16),
        w_ref[...].astype(jnp.bfloat16),
        dimension_numbers=(((1,), (1,)), ((), ())),
        preferred_element_type=jnp.float32,
    )
    y = y + b_ref[...]
    o_ref[...] = y * jax.nn.sigmoid(y)


def kernel(x, feature1, feature2, edge_index, params):
    del feature1, feature2, edge_index
    n, h = x.shape
    w = params["lin_w"]
    b = params["lin_b"].reshape(1, h)
    block = min(_BLOCK_ROWS, n)
    return pl.pallas_call(
        _lin_swish_kernel,
        grid=(pl.cdiv(n, block),),
        in_specs=[
            pl.BlockSpec((block, h), lambda i: (i, 0)),
            pl.BlockSpec((h, h), lambda i: (0, 0)),
            pl.BlockSpec((1, h), lambda i: (0, 0)),
        ],
        out_specs=pl.BlockSpec((block, h), lambda i: (i, 0)),
        out_shape=jax.ShapeDtypeStruct((n, h), jnp.float32),
        compiler_params=pltpu.CompilerParams(
            dimension_semantics=("arbitrary",),
        ),
    )(x, w, b)
